# baseline (speedup 1.0000x reference)
import jax
import jax.numpy as jnp
from jax.experimental import pallas as pl

B, N, M, K, GK = 4, 4096, 3072, 64, 8


def _sph_to_cart(theta, phi):
    st = jnp.sin(theta)
    return jnp.stack([st * jnp.cos(phi), st * jnp.sin(phi), jnp.cos(theta)], axis=-1)


def _head_mlp_kernel(g_ref, w1_ref, b1_ref, w2_ref, b2_ref, o_ref):
    h = jnp.maximum(jnp.dot(g_ref[...], w1_ref[...], preferred_element_type=jnp.float32) + b1_ref[...], 0.0)
    o_ref[...] = jnp.dot(h, w2_ref[...], preferred_element_type=jnp.float32) + b2_ref[...]


def kernel(xxx, pix_theta_phi, att_W1, att_b1, att_W2, att_b2, Wp, bp, Wrel0, brel0, Wroot0, Wrel1, brel1, Wroot1, Wrel2, brel2, Wroot2, Wo1, bo1, Wo2, bo2):
    los_tp = jnp.transpose(xxx[:, :2, :], (0, 2, 1))
    x = xxx[:, 2, :][:, :, None]
    r_los = _sph_to_cart(los_tp[..., 0], los_tp[..., 1])
    r_pix = _sph_to_cart(pix_theta_phi[:, 0], pix_theta_phi[:, 1])
    cosm = jnp.clip(jnp.einsum('bnc,mc->bnm', r_los, r_pix), -1.0, 1.0)
    cos_p = jnp.transpose(cosm, (0, 2, 1))
    topk_vals, topk_idx = jax.lax.top_k(cos_p, K)
    x_g = jax.vmap(lambda xb, idx: xb[idx])(x, topk_idx)
    d = jnp.arccos(jnp.clip(topk_vals, -1.0, 1.0))[..., None]
    feat = jnp.concatenate([x_g, d], axis=-1)
    h1 = jax.nn.relu(feat @ att_W1 + att_b1)
    logits = (h1 @ att_W2 + att_b2)[..., 0]
    logits = logits - jnp.max(logits, axis=2, keepdims=True)
    w = jax.nn.softmax(logits, axis=2)
    pooled = jnp.sum(w[..., None] * x_g, axis=2)
    h = jax.nn.relu(pooled @ Wp + bp)
    h_flat = h.reshape(B * M, -1)
    d2 = jnp.sum((r_pix[:, None, :] - r_pix[None, :, :]) ** 2, axis=-1)
    d2 = jnp.where(jnp.eye(M, dtype=bool), jnp.inf, d2)
    _, nbr = jax.lax.top_k(-d2, GK)
    src_local = nbr.reshape(-1)
    dst_local = jnp.repeat(jnp.arange(M), GK)
    offs = (jnp.arange(B) * M)[:, None]
    src = (src_local[None, :] + offs).reshape(-1)
    dst = (dst_local[None, :] + offs).reshape(-1)
    for Wrel, brel, Wroot in ((Wrel0, brel0, Wroot0), (Wrel1, brel1, Wroot1), (Wrel2, brel2, Wroot2)):
        agg = jax.ops.segment_sum(h_flat[src], dst, num_segments=B * M)
        h_flat = jax.nn.relu(agg @ Wrel + brel + h_flat @ Wroot)
    graph_feat = h_flat.reshape(B, M, -1).mean(axis=1)

    out = pl.pallas_call(
        _head_mlp_kernel,
        out_shape=jax.ShapeDtypeStruct((B, 200), jnp.float32),
    )(graph_feat, Wo1, bo1, Wo2, bo2)
    return out


# speedup vs baseline: 6.8416x; 6.8416x over previous
import functools

import jax
import jax.numpy as jnp
import numpy as np
from jax import lax
from jax.experimental import pallas as pl
from jax.experimental.pallas import tpu as pltpu
from jax.experimental.pallas import tpu_sc as plsc

B, N, M, K, GK = 4, 4096, 3072, 64, 8
BM = B * M

_IP = False

MB_A = 256
_S_LO = np.int32(np.uint32(0xBF800000 ^ 0x7FFFFFFF))
_S_HI = np.int32(0x3F800000 + 1)


def _bf(v):
    return v.astype(jnp.bfloat16).astype(jnp.float32)


def _stage_a_body(rp_ref, rl_ref, cos_ref, thr_ref):
    rp0 = _bf(rp_ref[:, 0:1])
    rp1 = _bf(rp_ref[:, 1:2])
    rp2 = _bf(rp_ref[:, 2:3])
    rl0 = _bf(rl_ref[0, 0:1, :])
    rl1 = _bf(rl_ref[0, 1:2, :])
    rl2 = _bf(rl_ref[0, 2:3, :])
    cos = jnp.clip(rp0 * rl0 + rp1 * rl1 + rp2 * rl2, -1.0, 1.0)
    cos_ref[0] = cos
    bits = lax.bitcast_convert_type(cos, jnp.int32)
    s = bits ^ (lax.shift_right_arithmetic(bits, 31) & jnp.int32(0x7FFFFFFF))

    def step(_, c):
        lo, hi = c
        mid = lax.shift_right_arithmetic(lo + hi, 1)
        cnt = jnp.sum(jnp.where(s >= mid, jnp.int32(1), jnp.int32(0)),
                      axis=1, keepdims=True)
        ge = cnt >= K
        return jnp.where(ge, mid, lo), jnp.where(ge, hi, mid)

    lo0 = jnp.full((MB_A, 1), _S_LO, jnp.int32)
    hi0 = jnp.full((MB_A, 1), _S_HI, jnp.int32)
    lo, _ = lax.fori_loop(0, 31, step, (lo0, hi0))
    tb = jnp.where(lo >= 0, lo, lo ^ jnp.int32(0x7FFFFFFF))
    thr_ref[0] = lax.bitcast_convert_type(tb, jnp.float32)


def _stage_a(r_pix, r_losT):
    return pl.pallas_call(
        _stage_a_body,
        grid=(B, M // MB_A),
        in_specs=[
            pl.BlockSpec((MB_A, 3), lambda b, j: (j, 0)),
            pl.BlockSpec((1, 3, N), lambda b, j: (b, 0, 0)),
        ],
        out_specs=[
            pl.BlockSpec((1, MB_A, N), lambda b, j: (b, j, 0)),
            pl.BlockSpec((1, MB_A, 1), lambda b, j: (b, j, 0)),
        ],
        out_shape=[
            jax.ShapeDtypeStruct((B, M, N), jnp.float32),
            jax.ShapeDtypeStruct((B, M, 1), jnp.float32),
        ],
        interpret=_IP,
    )(r_pix, r_losT)


_NC, _NS = 2, 16
_NW = _NC * _NS
_RPW = BM // _NW
_WPB = M // _RPW
_SEG = N // 16


def _stage_b_body(cos_hbm, thr_hbm, x_hbm, vals_hbm, xg_hbm,
                  rowbuf, xbuf, thrbuf, lanebuf, gebuf, idx64, val64, xg64):
    cid = lax.axis_index("c")
    sid = lax.axis_index("s")
    wid = sid * _NC + cid
    base_row = wid * _RPW
    b = wid // _WPB
    pltpu.sync_copy(x_hbm.at[pl.ds(b * N, N)], xbuf)
    pltpu.sync_copy(thr_hbm.at[pl.ds(base_row, _RPW)], thrbuf)
    iota = lax.iota(jnp.int32, 16)
    lane_base = iota * _SEG

    def row_fn(r, _carry):
        g = base_row + r
        pltpu.sync_copy(cos_hbm.at[pl.ds(g * N, N)], rowbuf)
        t = plsc.load_gather(thrbuf, [jnp.full((16,), r, jnp.int32)])

        def scan_fn(j, cnt):
            nidx = lane_base + j
            v = plsc.load_gather(rowbuf, [nidx])
            m = v >= t
            pos = jnp.where(m, lane_base + cnt, 0)
            plsc.store_scatter(lanebuf, [pos], nidx, mask=m)
            return cnt + jnp.where(m, 1, 0)

        cnt = lax.fori_loop(0, _SEG, scan_fn, jnp.zeros((16,), jnp.int32),
                            unroll=4)
        excl = plsc.cumsum(cnt) - cnt
        maxc = jnp.max(cnt)

        def merge_fn(c, _):
            mk = c < cnt
            entry = plsc.load_gather(lanebuf, [lane_base + c])
            entry = jnp.where(mk, entry, 0)
            pos = jnp.where(mk, excl + c, 0)
            plsc.store_scatter(gebuf, [pos], entry, mask=mk)
            return 0

        lax.fori_loop(0, maxc, merge_fn, 0)
        cge = jnp.sum(cnt)
        nch = (cge + 15) // 16

        def cgt_fn(c, acc):
            valid = (iota + c * 16) < cge
            e = jnp.where(valid, gebuf[pl.ds(c * 16, 16)], 0)
            ev = plsc.load_gather(rowbuf, [e])
            mg = (ev > t) & valid
            return acc + jnp.where(mg, 1, 0)

        cgt = jnp.sum(lax.fori_loop(0, nch, cgt_fn,
                                    jnp.zeros((16,), jnp.int32)))

        def fill_fn(c, carry):
            offgt, offeq = carry
            valid = (iota + c * 16) < cge
            e = jnp.where(valid, gebuf[pl.ds(c * 16, 16)], 0)
            ev = plsc.load_gather(rowbuf, [e])
            gt = ev > t
            mg = gt & valid
            me = (~gt) & valid
            pcg = plsc.cumsum(jnp.where(mg, 1, 0))
            pce = plsc.cumsum(jnp.where(me, 1, 0))
            posg = jnp.where(mg, offgt + pcg - 1, 0)
            mke = me & ((cgt + offeq + pce - 1) < K)
            pose = jnp.where(mke, cgt + offeq + pce - 1, 0)
            plsc.store_scatter(idx64, [posg], e, mask=mg)
            plsc.store_scatter(idx64, [pose], e, mask=mke)
            return offgt + jnp.sum(jnp.where(mg, 1, 0)), \
                   offeq + jnp.sum(jnp.where(me, 1, 0))

        lax.fori_loop(0, nch, fill_fn, (jnp.int32(0), jnp.int32(0)))

        for c4 in range(K // 16):
            iv = idx64[pl.ds(c4 * 16, 16)]
            val64[pl.ds(c4 * 16, 16)] = plsc.load_gather(rowbuf, [iv])
            xg64[pl.ds(c4 * 16, 16)] = plsc.load_gather(xbuf, [iv])
        pltpu.sync_copy(val64, vals_hbm.at[pl.ds(g * K, K)])
        pltpu.sync_copy(xg64, xg_hbm.at[pl.ds(g * K, K)])
        return 0

    lax.fori_loop(0, _RPW, row_fn, 0)


def _stage_b_emulate(cos_flat, thr_flat, x_flat):
    cos = cos_flat.reshape(BM, N)
    t = thr_flat.reshape(BM, 1)
    mask_gt = cos > t
    mask_eq = cos == t
    fill = K - jnp.sum(mask_gt, axis=1, keepdims=True)
    eqrank = jnp.cumsum(mask_eq.astype(jnp.int32), axis=1)
    sel = mask_gt | (mask_eq & (eqrank <= fill))
    idx = jnp.argsort(jnp.where(sel, 0, 1), axis=1, stable=True)[:, :K]
    vals = jnp.take_along_axis(cos, idx, axis=1)
    x = x_flat.reshape(B, N)
    xg = jax.vmap(lambda xb, ib: xb[ib])(x, idx.reshape(B, M * K))
    return vals.reshape(BM * K), xg.reshape(BM * K)


def _stage_b(cos_flat, thr_flat, x_flat):
    if _IP:
        return _stage_b_emulate(cos_flat, thr_flat, x_flat)
    mesh = plsc.VectorSubcoreMesh(core_axis_name="c", subcore_axis_name="s",
                                  num_cores=_NC, num_subcores=_NS)
    f = pl.kernel(
        _stage_b_body,
        out_type=(jax.ShapeDtypeStruct((BM * K,), jnp.float32),
                  jax.ShapeDtypeStruct((BM * K,), jnp.float32)),
        mesh=mesh,
        scratch_types=[
            pltpu.VMEM((N,), jnp.float32),
            pltpu.VMEM((N,), jnp.float32),
            pltpu.VMEM((_RPW,), jnp.float32),
            pltpu.VMEM((N,), jnp.int32),
            pltpu.VMEM((N + 16,), jnp.int32),
            pltpu.VMEM((K + 16,), jnp.int32),
            pltpu.VMEM((K,), jnp.float32),
            pltpu.VMEM((K,), jnp.float32),
        ],
        compiler_params=pltpu.CompilerParams(needs_layout_passes=False),
        interpret=_IP,
    )
    return f(cos_flat, thr_flat, x_flat)


MB_C = 512


def _stage_c_body(vals_ref, xg_ref, w1_ref, b1_ref, w2_ref, wp_ref, bp_ref,
                  h_ref):
    vals = vals_ref[...]
    xg = xg_ref[...]
    z = jnp.clip(vals, -1.0, 1.0)
    d = jnp.arctan2(jnp.sqrt(jnp.maximum((1.0 - z) * (1.0 + z), 0.0)), z)
    w1 = w1_ref[...]
    b1 = b1_ref[...]
    w2 = w2_ref[...]
    xgb = _bf(xg)
    db = _bf(d)
    w1b = _bf(w1)
    w2b = _bf(w2)
    logits = jnp.zeros_like(xg)
    for j in range(128):
        hj = jnp.maximum(xgb * w1b[0, j] + db * w1b[1, j] + b1[0, j], 0.0)
        logits = logits + _bf(hj) * w2b[0, j]
    lmax = jnp.max(logits, axis=1, keepdims=True)
    e = jnp.exp(logits - lmax)
    w = e / jnp.sum(e, axis=1, keepdims=True)
    pooled = jnp.sum(w * xg, axis=1, keepdims=True)
    h_ref[...] = jnp.maximum(pooled * wp_ref[...] + bp_ref[...], 0.0)


def _stage_c(vals, xg, att_W1, att_b1, att_W2, att_b2, Wp, bp):
    del att_b2
    b1 = att_b1.reshape(1, 128)
    return pl.pallas_call(
        _stage_c_body,
        grid=(BM // MB_C,),
        in_specs=[
            pl.BlockSpec((MB_C, K), lambda i: (i, 0)),
            pl.BlockSpec((MB_C, K), lambda i: (i, 0)),
            pl.BlockSpec((2, 128), lambda i: (0, 0)),
            pl.BlockSpec((1, 128), lambda i: (0, 0)),
            pl.BlockSpec((1, 128), lambda i: (0, 0)),
            pl.BlockSpec((1, 256), lambda i: (0, 0)),
            pl.BlockSpec((1, 256), lambda i: (0, 0)),
        ],
        out_specs=pl.BlockSpec((MB_C, 256), lambda i: (i, 0)),
        out_shape=jax.ShapeDtypeStruct((BM, 256), jnp.float32),
        interpret=_IP,
    )(vals, xg, att_W1, b1, att_W2.reshape(1, 128), Wp, bp.reshape(1, 256))


MB_D = 256
_BIGI = np.int32(2**30)


def _stage_d_body(pix_ref, pixt_ref, a_ref):
    blk = pl.program_id(0)
    a0 = pix_ref[:, 0:1]
    a1 = pix_ref[:, 1:2]
    a2 = pix_ref[:, 2:3]
    c0 = pixt_ref[0:1, :]
    c1 = pixt_ref[1:2, :]
    c2 = pixt_ref[2:3, :]
    d2 = (a0 - c0) ** 2 + (a1 - c1) ** 2 + (a2 - c2) ** 2
    rows = blk * MB_D + lax.broadcasted_iota(jnp.int32, (MB_D, M), 0)
    cols = lax.broadcasted_iota(jnp.int32, (MB_D, M), 1)
    d2 = jnp.where(rows == cols, jnp.inf, d2)
    nd2 = -d2
    a = jnp.zeros((MB_D, M), jnp.float32)
    for _ in range(GK):
        vmax = jnp.max(nd2, axis=1, keepdims=True)
        cand = jnp.where(nd2 == vmax, cols, _BIGI)
        imin = jnp.min(cand, axis=1, keepdims=True)
        hit = cols == imin
        a = a + jnp.where(hit, 1.0, 0.0)
        nd2 = jnp.where(hit, -jnp.inf, nd2)
    a_ref[...] = a.astype(jnp.bfloat16)


def _stage_d(pix, pixt):
    return pl.pallas_call(
        _stage_d_body,
        grid=(M // MB_D,),
        in_specs=[
            pl.BlockSpec((MB_D, 3), lambda i: (i, 0)),
            pl.BlockSpec((3, M), lambda i: (0, 0)),
        ],
        out_specs=pl.BlockSpec((MB_D, M), lambda i: (i, 0)),
        out_shape=jax.ShapeDtypeStruct((M, M), jnp.bfloat16),
        interpret=_IP,
    )(pix, pixt)


MB_E = 1024


def _gnn_layer_body(hfull_ref, hrows_ref, a_ref, wrel_ref, brel_ref,
                    wroot_ref, out_ref):
    h = hfull_ref[0]
    a = a_ref[...]
    hi = h.astype(jnp.bfloat16)
    lo = (h - hi.astype(jnp.float32)).astype(jnp.bfloat16)
    agg = (jnp.dot(a, hi, preferred_element_type=jnp.float32) +
           jnp.dot(a, lo, preferred_element_type=jnp.float32))
    out_ref[0] = jnp.maximum(
        jnp.dot(agg.astype(jnp.bfloat16), wrel_ref[...].astype(jnp.bfloat16),
                preferred_element_type=jnp.float32)
        + brel_ref[...]
        + jnp.dot(hrows_ref[0].astype(jnp.bfloat16),
                  wroot_ref[...].astype(jnp.bfloat16),
                  preferred_element_type=jnp.float32),
        0.0)


def _gnn_layer(h, a, wrel, brel, wroot):
    return pl.pallas_call(
        _gnn_layer_body,
        grid=(B, M // MB_E),
        in_specs=[
            pl.BlockSpec((1, M, 256), lambda b, j: (b, 0, 0)),
            pl.BlockSpec((1, MB_E, 256), lambda b, j: (b, j, 0)),
            pl.BlockSpec((MB_E, M), lambda b, j: (j, 0)),
            pl.BlockSpec((256, 256), lambda b, j: (0, 0)),
            pl.BlockSpec((1, 256), lambda b, j: (0, 0)),
            pl.BlockSpec((256, 256), lambda b, j: (0, 0)),
        ],
        out_specs=pl.BlockSpec((1, MB_E, 256), lambda b, j: (b, j, 0)),
        out_shape=jax.ShapeDtypeStruct((B, M, 256), jnp.float32),
        interpret=_IP,
    )(h, h, a, wrel, brel, wroot)


def _head_body(h_ref, wo1_ref, bo1_ref, wo2_ref, bo2_ref, out_ref):
    gf = jnp.mean(h_ref[0], axis=0, keepdims=True)
    g1 = jnp.maximum(
        jnp.dot(gf.astype(jnp.bfloat16), wo1_ref[...].astype(jnp.bfloat16),
                preferred_element_type=jnp.float32)
        + bo1_ref[...], 0.0)
    out_ref[0] = (jnp.dot(g1.astype(jnp.bfloat16),
                          wo2_ref[...].astype(jnp.bfloat16),
                          preferred_element_type=jnp.float32)
                  + bo2_ref[...])


def _stage_e(h, a, wrel, brel, wroot, Wo1, bo1, Wo2, bo2):
    for l in range(3):
        h = _gnn_layer(h, a, wrel[l], brel[l].reshape(1, 256), wroot[l])
    return pl.pallas_call(
        _head_body,
        grid=(B,),
        in_specs=[
            pl.BlockSpec((1, M, 256), lambda b: (b, 0, 0)),
            pl.BlockSpec((256, 256), lambda b: (0, 0)),
            pl.BlockSpec((1, 256), lambda b: (0, 0)),
            pl.BlockSpec((256, 200), lambda b: (0, 0)),
            pl.BlockSpec((1, 200), lambda b: (0, 0)),
        ],
        out_specs=pl.BlockSpec((1, 1, 200), lambda b: (b, 0, 0)),
        out_shape=jax.ShapeDtypeStruct((B, 1, 200), jnp.float32),
        interpret=_IP,
    )(h, Wo1, bo1, Wo2, bo2)


def kernel(xxx, pix_theta_phi, att_W1, att_b1, att_W2, att_b2, Wp, bp,
           Wrel0, brel0, Wroot0, Wrel1, brel1, Wroot1, Wrel2, brel2, Wroot2,
           Wo1, bo1, Wo2, bo2):
    th_l = xxx[:, 0, :]
    ph_l = xxx[:, 1, :]
    st_l = jnp.sin(th_l)
    r_losT = jnp.stack([st_l * jnp.cos(ph_l), st_l * jnp.sin(ph_l),
                        jnp.cos(th_l)], axis=1)
    st_p = jnp.sin(pix_theta_phi[:, 0])
    r_pix = jnp.stack([st_p * jnp.cos(pix_theta_phi[:, 1]),
                       st_p * jnp.sin(pix_theta_phi[:, 1]),
                       jnp.cos(pix_theta_phi[:, 0])], axis=-1)

    cos, thr = _stage_a(r_pix, r_losT)
    x_flat = xxx[:, 2, :].reshape(B * N)
    vals_f, xg_f = _stage_b(cos.reshape(B * M * N), thr.reshape(BM), x_flat)
    vals = vals_f.reshape(BM, K)
    xg = xg_f.reshape(BM, K)
    h = _stage_c(vals, xg, att_W1, att_b1, att_W2, att_b2, Wp, bp)
    a = _stage_d(r_pix, r_pix.T)
    wrel = jnp.stack([Wrel0, Wrel1, Wrel2])
    brel = jnp.stack([brel0, brel1, brel2])
    wroot = jnp.stack([Wroot0, Wroot1, Wroot2])
    out = _stage_e(h.reshape(B, M, 256), a, wrel, brel, wroot,
                   Wo1, bo1.reshape(1, 256), Wo2, bo2.reshape(1, 200))
    return out.reshape(B, 200)


# speedup vs baseline: 7.4781x; 1.0930x over previous
import functools

import jax
import jax.numpy as jnp
import numpy as np
from jax import lax
from jax.experimental import pallas as pl
from jax.experimental.pallas import tpu as pltpu
from jax.experimental.pallas import tpu_sc as plsc

B, N, M, K, GK = 4, 4096, 3072, 64, 8
BM = B * M

_IP = False

MB_A = 256
_S_LO = np.int32(np.uint32(0xBF800000 ^ 0x7FFFFFFF))
_S_HI = np.int32(0x3F800000 + 1)


def _bf(v):
    return v.astype(jnp.bfloat16).astype(jnp.float32)


def _stage_a_body(rp_ref, rl_ref, cos_ref, thr_ref):
    rp0 = _bf(rp_ref[:, 0:1])
    rp1 = _bf(rp_ref[:, 1:2])
    rp2 = _bf(rp_ref[:, 2:3])
    rl0 = _bf(rl_ref[0, 0:1, :])
    rl1 = _bf(rl_ref[0, 1:2, :])
    rl2 = _bf(rl_ref[0, 2:3, :])
    cos = jnp.clip(rp0 * rl0 + rp1 * rl1 + rp2 * rl2, -1.0, 1.0)
    cos_ref[...] = cos
    bits = lax.bitcast_convert_type(cos, jnp.int32)
    s = bits ^ (lax.shift_right_arithmetic(bits, 31) & jnp.int32(0x7FFFFFFF))

    def step(_, c):
        lo, hi = c
        mid = lax.shift_right_arithmetic(lo + hi, 1)
        cnt = jnp.sum(jnp.where(s >= mid, jnp.int32(1), jnp.int32(0)),
                      axis=1, keepdims=True)
        ge = cnt >= K
        return jnp.where(ge, mid, lo), jnp.where(ge, hi, mid)

    lo0 = jnp.full((MB_A, 1), _S_LO, jnp.int32)
    hi0 = jnp.full((MB_A, 1), _S_HI, jnp.int32)
    lo, _ = lax.fori_loop(0, 31, step, (lo0, hi0))
    tb = jnp.where(lo >= 0, lo, lo ^ jnp.int32(0x7FFFFFFF))
    thr_ref[0] = lax.bitcast_convert_type(tb, jnp.float32)


def _stage_a(r_pix, r_losT):
    return pl.pallas_call(
        _stage_a_body,
        grid=(B, M // MB_A),
        in_specs=[
            pl.BlockSpec((MB_A, 3), lambda b, j: (j, 0)),
            pl.BlockSpec((1, 3, N), lambda b, j: (b, 0, 0)),
        ],
        out_specs=[
            pl.BlockSpec((MB_A, N), lambda b, j: (b * (M // MB_A) + j, 0)),
            pl.BlockSpec((1, MB_A, 1), lambda b, j: (b, j, 0)),
        ],
        out_shape=[
            jax.ShapeDtypeStruct((BM, N), jnp.float32),
            jax.ShapeDtypeStruct((B, M, 1), jnp.float32),
        ],
        interpret=_IP,
    )(r_pix, r_losT)


_NC, _NS = 2, 16
_NW = _NC * _NS
_RPW = BM // _NW
_WPB = M // _RPW
_SEG = N // 16


_RB = 8
_NBLK = _RPW // _RB


def _stage_b_body(cos_hbm, thr_hbm, x_hbm, vals_hbm, xg_hbm,
                  rowbuf0, rowbuf1, xbuf, thrbuf, lanebuf, gebuf, idx64,
                  val8, xg8, sem0, sem1):
    cid = lax.axis_index("c")
    sid = lax.axis_index("s")
    wid = sid * _NC + cid
    base_row = wid * _RPW
    b = wid // _WPB
    pltpu.sync_copy(x_hbm.at[pl.ds(b * N, N)], xbuf)
    pltpu.sync_copy(thr_hbm.at[pl.ds(base_row, _RPW)], thrbuf)
    iota = lax.iota(jnp.int32, 16)
    lane_base = iota * _SEG

    def blk_src(blk):
        return cos_hbm.at[pl.ds(base_row + blk * _RB, _RB), :]

    pltpu.async_copy(blk_src(0), rowbuf0, sem0)
    pltpu.async_copy(blk_src(1), rowbuf1, sem1)

    def process_block(blk, buf, sem):
        pltpu.make_async_copy(blk_src(blk), buf, sem).wait()
        g0 = base_row + blk * _RB

        def row_fn(i, _carry):
            r = blk * _RB + i
            rsp = jnp.full((16,), i, jnp.int32)
            t = plsc.load_gather(thrbuf, [jnp.full((16,), r, jnp.int32)])

            def scan_fn(j, cnt):
                nidx = lane_base + j
                v = plsc.load_gather(buf, [rsp, nidx])
                m = v >= t
                pos = jnp.where(m, lane_base + cnt, 0)
                plsc.store_scatter(lanebuf, [pos], nidx, mask=m)
                return cnt + jnp.where(m, 1, 0)

            cnt = lax.fori_loop(0, _SEG, scan_fn, jnp.zeros((16,), jnp.int32),
                                unroll=4)
            excl = plsc.cumsum(cnt) - cnt
            maxc = jnp.max(cnt)

            def merge_fn(c, _):
                mk = c < cnt
                entry = plsc.load_gather(lanebuf, [lane_base + c])
                entry = jnp.where(mk, entry, 0)
                pos = jnp.where(mk, excl + c, 0)
                plsc.store_scatter(gebuf, [pos], entry, mask=mk)
                return 0

            lax.fori_loop(0, maxc, merge_fn, 0)
            cge = jnp.sum(cnt)
            nch = (cge + 15) // 16

            def cgt_fn(c, acc):
                valid = (iota + c * 16) < cge
                e = jnp.where(valid, gebuf[pl.ds(c * 16, 16)], 0)
                ev = plsc.load_gather(buf, [rsp, e])
                mg = (ev > t) & valid
                return acc + jnp.where(mg, 1, 0)

            cgt = jnp.sum(lax.fori_loop(0, nch, cgt_fn,
                                        jnp.zeros((16,), jnp.int32)))

            def fill_fn(c, carry):
                offgt, offeq = carry
                valid = (iota + c * 16) < cge
                e = jnp.where(valid, gebuf[pl.ds(c * 16, 16)], 0)
                ev = plsc.load_gather(buf, [rsp, e])
                gt = ev > t
                mg = gt & valid
                me = (~gt) & valid
                pcg = plsc.cumsum(jnp.where(mg, 1, 0))
                pce = plsc.cumsum(jnp.where(me, 1, 0))
                posg = jnp.where(mg, offgt + pcg - 1, 0)
                mke = me & ((cgt + offeq + pce - 1) < K)
                pose = jnp.where(mke, cgt + offeq + pce - 1, 0)
                plsc.store_scatter(idx64, [posg], e, mask=mg)
                plsc.store_scatter(idx64, [pose], e, mask=mke)
                return offgt + jnp.sum(jnp.where(mg, 1, 0)), \
                       offeq + jnp.sum(jnp.where(me, 1, 0))

            lax.fori_loop(0, nch, fill_fn, (jnp.int32(0), jnp.int32(0)))

            for c4 in range(K // 16):
                iv = idx64[pl.ds(c4 * 16, 16)]
                val8[pl.ds(i * K + c4 * 16, 16)] = \
                    plsc.load_gather(buf, [rsp, iv])
                xg8[pl.ds(i * K + c4 * 16, 16)] = plsc.load_gather(xbuf, [iv])
            return 0

        lax.fori_loop(0, _RB, row_fn, 0)
        pltpu.sync_copy(val8, vals_hbm.at[pl.ds(g0 * K, _RB * K)])
        pltpu.sync_copy(xg8, xg_hbm.at[pl.ds(g0 * K, _RB * K)])

    def pair_fn(p, _carry):
        blk0 = p * 2
        process_block(blk0, rowbuf0, sem0)

        @pl.when(blk0 + 2 < _NBLK)
        def _():
            pltpu.async_copy(blk_src(blk0 + 2), rowbuf0, sem0)

        process_block(blk0 + 1, rowbuf1, sem1)

        @pl.when(blk0 + 3 < _NBLK)
        def _():
            pltpu.async_copy(blk_src(blk0 + 3), rowbuf1, sem1)

        return 0

    lax.fori_loop(0, _NBLK // 2, pair_fn, 0)


def _stage_b_emulate(cos_flat, thr_flat, x_flat):
    cos = cos_flat.reshape(BM, N)
    t = thr_flat.reshape(BM, 1)
    mask_gt = cos > t
    mask_eq = cos == t
    fill = K - jnp.sum(mask_gt, axis=1, keepdims=True)
    eqrank = jnp.cumsum(mask_eq.astype(jnp.int32), axis=1)
    sel = mask_gt | (mask_eq & (eqrank <= fill))
    idx = jnp.argsort(jnp.where(sel, 0, 1), axis=1, stable=True)[:, :K]
    vals = jnp.take_along_axis(cos, idx, axis=1)
    x = x_flat.reshape(B, N)
    xg = jax.vmap(lambda xb, ib: xb[ib])(x, idx.reshape(B, M * K))
    return vals.reshape(BM * K), xg.reshape(BM * K)


def _stage_b(cos2d, thr_flat, x_flat):
    if _IP:
        return _stage_b_emulate(cos2d.reshape(BM * N), thr_flat, x_flat)
    mesh = plsc.VectorSubcoreMesh(core_axis_name="c", subcore_axis_name="s",
                                  num_cores=_NC, num_subcores=_NS)
    f = pl.kernel(
        _stage_b_body,
        out_type=(jax.ShapeDtypeStruct((BM * K,), jnp.float32),
                  jax.ShapeDtypeStruct((BM * K,), jnp.float32)),
        mesh=mesh,
        scratch_types=[
            pltpu.VMEM((_RB, N), jnp.float32),
            pltpu.VMEM((_RB, N), jnp.float32),
            pltpu.VMEM((N,), jnp.float32),
            pltpu.VMEM((_RPW,), jnp.float32),
            pltpu.VMEM((N,), jnp.int32),
            pltpu.VMEM((N + 16,), jnp.int32),
            pltpu.VMEM((K + 16,), jnp.int32),
            pltpu.VMEM((_RB * K,), jnp.float32),
            pltpu.VMEM((_RB * K,), jnp.float32),
            pltpu.SemaphoreType.DMA,
            pltpu.SemaphoreType.DMA,
        ],
        compiler_params=pltpu.CompilerParams(needs_layout_passes=False),
        interpret=_IP,
    )
    return f(cos2d, thr_flat, x_flat)


MB_C = 512


def _stage_c_body(vals_ref, xg_ref, w1_ref, b1_ref, w2_ref, wp_ref, bp_ref,
                  h_ref):
    vals = vals_ref[...]
    xg = xg_ref[...]
    z = jnp.clip(vals, -1.0, 1.0)
    d = jnp.arctan2(jnp.sqrt(jnp.maximum((1.0 - z) * (1.0 + z), 0.0)), z)
    w1 = w1_ref[...]
    b1 = b1_ref[...]
    w2 = w2_ref[...]
    xgb = _bf(xg)
    db = _bf(d)
    w1b = _bf(w1)
    w2b = _bf(w2)
    logits = jnp.zeros_like(xg)
    for j in range(128):
        hj = jnp.maximum(xgb * w1b[0, j] + db * w1b[1, j] + b1[0, j], 0.0)
        logits = logits + _bf(hj) * w2b[0, j]
    lmax = jnp.max(logits, axis=1, keepdims=True)
    e = jnp.exp(logits - lmax)
    w = e / jnp.sum(e, axis=1, keepdims=True)
    pooled = jnp.sum(w * xg, axis=1, keepdims=True)
    h_ref[...] = jnp.maximum(pooled * wp_ref[...] + bp_ref[...], 0.0)


def _stage_c(vals, xg, att_W1, att_b1, att_W2, att_b2, Wp, bp):
    del att_b2
    b1 = att_b1.reshape(1, 128)
    return pl.pallas_call(
        _stage_c_body,
        grid=(BM // MB_C,),
        in_specs=[
            pl.BlockSpec((MB_C, K), lambda i: (i, 0)),
            pl.BlockSpec((MB_C, K), lambda i: (i, 0)),
            pl.BlockSpec((2, 128), lambda i: (0, 0)),
            pl.BlockSpec((1, 128), lambda i: (0, 0)),
            pl.BlockSpec((1, 128), lambda i: (0, 0)),
            pl.BlockSpec((1, 256), lambda i: (0, 0)),
            pl.BlockSpec((1, 256), lambda i: (0, 0)),
        ],
        out_specs=pl.BlockSpec((MB_C, 256), lambda i: (i, 0)),
        out_shape=jax.ShapeDtypeStruct((BM, 256), jnp.float32),
        interpret=_IP,
    )(vals, xg, att_W1, b1, att_W2.reshape(1, 128), Wp, bp.reshape(1, 256))


MB_D = 256
_BIGI = np.int32(2**30)


def _stage_d_body(pix_ref, pixt_ref, a_ref):
    blk = pl.program_id(0)
    a0 = pix_ref[:, 0:1]
    a1 = pix_ref[:, 1:2]
    a2 = pix_ref[:, 2:3]
    c0 = pixt_ref[0:1, :]
    c1 = pixt_ref[1:2, :]
    c2 = pixt_ref[2:3, :]
    d2 = (a0 - c0) ** 2 + (a1 - c1) ** 2 + (a2 - c2) ** 2
    rows = blk * MB_D + lax.broadcasted_iota(jnp.int32, (MB_D, M), 0)
    cols = lax.broadcasted_iota(jnp.int32, (MB_D, M), 1)
    d2 = jnp.where(rows == cols, jnp.inf, d2)
    nd2 = -d2
    a = jnp.zeros((MB_D, M), jnp.float32)
    for _ in range(GK):
        vmax = jnp.max(nd2, axis=1, keepdims=True)
        cand = jnp.where(nd2 == vmax, cols, _BIGI)
        imin = jnp.min(cand, axis=1, keepdims=True)
        hit = cols == imin
        a = a + jnp.where(hit, 1.0, 0.0)
        nd2 = jnp.where(hit, -jnp.inf, nd2)
    a_ref[...] = a.astype(jnp.bfloat16)


def _stage_d(pix, pixt):
    return pl.pallas_call(
        _stage_d_body,
        grid=(M // MB_D,),
        in_specs=[
            pl.BlockSpec((MB_D, 3), lambda i: (i, 0)),
            pl.BlockSpec((3, M), lambda i: (0, 0)),
        ],
        out_specs=pl.BlockSpec((MB_D, M), lambda i: (i, 0)),
        out_shape=jax.ShapeDtypeStruct((M, M), jnp.bfloat16),
        interpret=_IP,
    )(pix, pixt)


MB_E = 1024


def _gnn_layer_body(hfull_ref, hrows_ref, a_ref, wrel_ref, brel_ref,
                    wroot_ref, out_ref):
    h = hfull_ref[0]
    a = a_ref[...]
    hi = h.astype(jnp.bfloat16)
    lo = (h - hi.astype(jnp.float32)).astype(jnp.bfloat16)
    agg = (jnp.dot(a, hi, preferred_element_type=jnp.float32) +
           jnp.dot(a, lo, preferred_element_type=jnp.float32))
    out_ref[0] = jnp.maximum(
        jnp.dot(agg.astype(jnp.bfloat16), wrel_ref[...].astype(jnp.bfloat16),
                preferred_element_type=jnp.float32)
        + brel_ref[...]
        + jnp.dot(hrows_ref[0].astype(jnp.bfloat16),
                  wroot_ref[...].astype(jnp.bfloat16),
                  preferred_element_type=jnp.float32),
        0.0)


def _gnn_layer(h, a, wrel, brel, wroot):
    return pl.pallas_call(
        _gnn_layer_body,
        grid=(B, M // MB_E),
        in_specs=[
            pl.BlockSpec((1, M, 256), lambda b, j: (b, 0, 0)),
            pl.BlockSpec((1, MB_E, 256), lambda b, j: (b, j, 0)),
            pl.BlockSpec((MB_E, M), lambda b, j: (j, 0)),
            pl.BlockSpec((256, 256), lambda b, j: (0, 0)),
            pl.BlockSpec((1, 256), lambda b, j: (0, 0)),
            pl.BlockSpec((256, 256), lambda b, j: (0, 0)),
        ],
        out_specs=pl.BlockSpec((1, MB_E, 256), lambda b, j: (b, j, 0)),
        out_shape=jax.ShapeDtypeStruct((B, M, 256), jnp.float32),
        interpret=_IP,
    )(h, h, a, wrel, brel, wroot)


def _head_body(h_ref, wo1_ref, bo1_ref, wo2_ref, bo2_ref, out_ref):
    gf = jnp.mean(h_ref[0], axis=0, keepdims=True)
    g1 = jnp.maximum(
        jnp.dot(gf.astype(jnp.bfloat16), wo1_ref[...].astype(jnp.bfloat16),
                preferred_element_type=jnp.float32)
        + bo1_ref[...], 0.0)
    out_ref[0] = (jnp.dot(g1.astype(jnp.bfloat16),
                          wo2_ref[...].astype(jnp.bfloat16),
                          preferred_element_type=jnp.float32)
                  + bo2_ref[...])


def _stage_e(h, a, wrel, brel, wroot, Wo1, bo1, Wo2, bo2):
    for l in range(3):
        h = _gnn_layer(h, a, wrel[l], brel[l].reshape(1, 256), wroot[l])
    return pl.pallas_call(
        _head_body,
        grid=(B,),
        in_specs=[
            pl.BlockSpec((1, M, 256), lambda b: (b, 0, 0)),
            pl.BlockSpec((256, 256), lambda b: (0, 0)),
            pl.BlockSpec((1, 256), lambda b: (0, 0)),
            pl.BlockSpec((256, 200), lambda b: (0, 0)),
            pl.BlockSpec((1, 200), lambda b: (0, 0)),
        ],
        out_specs=pl.BlockSpec((1, 1, 200), lambda b: (b, 0, 0)),
        out_shape=jax.ShapeDtypeStruct((B, 1, 200), jnp.float32),
        interpret=_IP,
    )(h, Wo1, bo1, Wo2, bo2)


def kernel(xxx, pix_theta_phi, att_W1, att_b1, att_W2, att_b2, Wp, bp,
           Wrel0, brel0, Wroot0, Wrel1, brel1, Wroot1, Wrel2, brel2, Wroot2,
           Wo1, bo1, Wo2, bo2):
    th_l = xxx[:, 0, :]
    ph_l = xxx[:, 1, :]
    st_l = jnp.sin(th_l)
    r_losT = jnp.stack([st_l * jnp.cos(ph_l), st_l * jnp.sin(ph_l),
                        jnp.cos(th_l)], axis=1)
    st_p = jnp.sin(pix_theta_phi[:, 0])
    r_pix = jnp.stack([st_p * jnp.cos(pix_theta_phi[:, 1]),
                       st_p * jnp.sin(pix_theta_phi[:, 1]),
                       jnp.cos(pix_theta_phi[:, 0])], axis=-1)

    cos, thr = _stage_a(r_pix, r_losT)
    x_flat = xxx[:, 2, :].reshape(B * N)
    vals_f, xg_f = _stage_b(cos, thr.reshape(BM), x_flat)
    vals = vals_f.reshape(BM, K)
    xg = xg_f.reshape(BM, K)
    h = _stage_c(vals, xg, att_W1, att_b1, att_W2, att_b2, Wp, bp)
    a = _stage_d(r_pix, r_pix.T)
    wrel = jnp.stack([Wrel0, Wrel1, Wrel2])
    brel = jnp.stack([brel0, brel1, brel2])
    wroot = jnp.stack([Wroot0, Wroot1, Wroot2])
    out = _stage_e(h.reshape(B, M, 256), a, wrel, brel, wroot,
                   Wo1, bo1.reshape(1, 256), Wo2, bo2.reshape(1, 200))
    return out.reshape(B, 200)


# speedup vs baseline: 7.6970x; 1.0293x over previous
import functools

import jax
import jax.numpy as jnp
import numpy as np
from jax import lax
from jax.experimental import pallas as pl
from jax.experimental.pallas import tpu as pltpu
from jax.experimental.pallas import tpu_sc as plsc

B, N, M, K, GK = 4, 4096, 3072, 64, 8
BM = B * M

_IP = False

MB_A = 256
_S_LO = np.int32(np.uint32(0xBF800000 ^ 0x7FFFFFFF))
_S_HI = np.int32(0x3F800000 + 1)


def _bf(v):
    return v.astype(jnp.bfloat16).astype(jnp.float32)


def _stage_a_body(rp_ref, rl_ref, cos_ref, thr_ref):
    rp0 = _bf(rp_ref[:, 0:1])
    rp1 = _bf(rp_ref[:, 1:2])
    rp2 = _bf(rp_ref[:, 2:3])
    rl0 = _bf(rl_ref[0, 0:1, :])
    rl1 = _bf(rl_ref[0, 1:2, :])
    rl2 = _bf(rl_ref[0, 2:3, :])
    cos = jnp.clip(rp0 * rl0 + rp1 * rl1 + rp2 * rl2, -1.0, 1.0)
    cos_ref[...] = cos
    bits = lax.bitcast_convert_type(cos, jnp.int32)
    s = bits ^ (lax.shift_right_arithmetic(bits, 31) & jnp.int32(0x7FFFFFFF))

    def step(_, c):
        lo, hi = c
        mid = lax.shift_right_arithmetic(lo + hi, 1)
        cnt = jnp.sum(jnp.where(s >= mid, jnp.int32(1), jnp.int32(0)),
                      axis=1, keepdims=True)
        ge = cnt >= K
        return jnp.where(ge, mid, lo), jnp.where(ge, hi, mid)

    lo0 = jnp.full((MB_A, 1), _S_LO, jnp.int32)
    hi0 = jnp.full((MB_A, 1), _S_HI, jnp.int32)
    lo, _ = lax.fori_loop(0, 31, step, (lo0, hi0))
    tb = jnp.where(lo >= 0, lo, lo ^ jnp.int32(0x7FFFFFFF))
    thr_ref[0] = lax.bitcast_convert_type(tb, jnp.float32)


def _stage_a(r_pix, r_losT):
    return pl.pallas_call(
        _stage_a_body,
        grid=(B, M // MB_A),
        in_specs=[
            pl.BlockSpec((MB_A, 3), lambda b, j: (j, 0)),
            pl.BlockSpec((1, 3, N), lambda b, j: (b, 0, 0)),
        ],
        out_specs=[
            pl.BlockSpec((MB_A, N), lambda b, j: (b * (M // MB_A) + j, 0)),
            pl.BlockSpec((1, MB_A, 1), lambda b, j: (b, j, 0)),
        ],
        out_shape=[
            jax.ShapeDtypeStruct((BM, N), jnp.float32),
            jax.ShapeDtypeStruct((B, M, 1), jnp.float32),
        ],
        interpret=_IP,
    )(r_pix, r_losT)


_NC, _NS = 2, 16
_NW = _NC * _NS
_RPW = BM // _NW
_WPB = M // _RPW
_SEG = N // 16


_RB = 8
_NBLK = _RPW // _RB


def _stage_b_body(cos_hbm, thr_hbm, x_hbm, vals_hbm, xg_hbm,
                  rowbuf0, rowbuf1, xbuf, thrbuf, lanebuf, gebuf, idx64,
                  val8, xg8, sem0, sem1):
    cid = lax.axis_index("c")
    sid = lax.axis_index("s")
    wid = sid * _NC + cid
    base_row = wid * _RPW
    b = wid // _WPB
    pltpu.sync_copy(x_hbm.at[pl.ds(b * N, N)], xbuf)
    pltpu.sync_copy(thr_hbm.at[pl.ds(base_row, _RPW)], thrbuf)
    iota = lax.iota(jnp.int32, 16)
    lane_base = iota * _SEG

    def blk_src(blk):
        return cos_hbm.at[pl.ds(base_row + blk * _RB, _RB), :]

    pltpu.async_copy(blk_src(0), rowbuf0, sem0)
    pltpu.async_copy(blk_src(1), rowbuf1, sem1)

    def process_block(blk, buf, sem):
        pltpu.make_async_copy(blk_src(blk), buf, sem).wait()
        g0 = base_row + blk * _RB

        def row_fn(i, _carry):
            r = blk * _RB + i
            rsp = jnp.full((16,), i, jnp.int32)
            t = plsc.load_gather(thrbuf, [jnp.full((16,), r, jnp.int32)])

            def scan_fn(j, cnt):
                nidx = lane_base + j
                v = plsc.load_gather(buf, [rsp, nidx])
                m = v >= t
                pos = jnp.where(m, lane_base + cnt, 0)
                plsc.store_scatter(lanebuf, [pos], nidx, mask=m)
                return cnt + jnp.where(m, 1, 0)

            cnt = lax.fori_loop(0, _SEG, scan_fn, jnp.zeros((16,), jnp.int32),
                                unroll=16)
            excl = plsc.cumsum(cnt) - cnt
            maxc = jnp.max(cnt)

            def merge_fn(c, _):
                mk = c < cnt
                entry = plsc.load_gather(lanebuf, [lane_base + c])
                entry = jnp.where(mk, entry, 0)
                pos = jnp.where(mk, excl + c, 0)
                plsc.store_scatter(gebuf, [pos], entry, mask=mk)
                return 0

            lax.fori_loop(0, maxc, merge_fn, 0)
            cge = jnp.sum(cnt)
            nch = (cge + 15) // 16

            def cgt_fn(c, acc):
                valid = (iota + c * 16) < cge
                e = jnp.where(valid, gebuf[pl.ds(c * 16, 16)], 0)
                ev = plsc.load_gather(buf, [rsp, e])
                mg = (ev > t) & valid
                return acc + jnp.where(mg, 1, 0)

            cgt = jnp.sum(lax.fori_loop(0, nch, cgt_fn,
                                        jnp.zeros((16,), jnp.int32)))

            def fill_fn(c, carry):
                offgt, offeq = carry
                valid = (iota + c * 16) < cge
                e = jnp.where(valid, gebuf[pl.ds(c * 16, 16)], 0)
                ev = plsc.load_gather(buf, [rsp, e])
                gt = ev > t
                mg = gt & valid
                me = (~gt) & valid
                pcg = plsc.cumsum(jnp.where(mg, 1, 0))
                pce = plsc.cumsum(jnp.where(me, 1, 0))
                posg = jnp.where(mg, offgt + pcg - 1, 0)
                mke = me & ((cgt + offeq + pce - 1) < K)
                pose = jnp.where(mke, cgt + offeq + pce - 1, 0)
                plsc.store_scatter(idx64, [posg], e, mask=mg)
                plsc.store_scatter(idx64, [pose], e, mask=mke)
                return offgt + jnp.sum(jnp.where(mg, 1, 0)), \
                       offeq + jnp.sum(jnp.where(me, 1, 0))

            lax.fori_loop(0, nch, fill_fn, (jnp.int32(0), jnp.int32(0)))

            for c4 in range(K // 16):
                iv = idx64[pl.ds(c4 * 16, 16)]
                val8[pl.ds(i * K + c4 * 16, 16)] = \
                    plsc.load_gather(buf, [rsp, iv])
                xg8[pl.ds(i * K + c4 * 16, 16)] = plsc.load_gather(xbuf, [iv])
            return 0

        lax.fori_loop(0, _RB, row_fn, 0)
        pltpu.sync_copy(val8, vals_hbm.at[pl.ds(g0 * K, _RB * K)])
        pltpu.sync_copy(xg8, xg_hbm.at[pl.ds(g0 * K, _RB * K)])

    def pair_fn(p, _carry):
        blk0 = p * 2
        process_block(blk0, rowbuf0, sem0)

        @pl.when(blk0 + 2 < _NBLK)
        def _():
            pltpu.async_copy(blk_src(blk0 + 2), rowbuf0, sem0)

        process_block(blk0 + 1, rowbuf1, sem1)

        @pl.when(blk0 + 3 < _NBLK)
        def _():
            pltpu.async_copy(blk_src(blk0 + 3), rowbuf1, sem1)

        return 0

    lax.fori_loop(0, _NBLK // 2, pair_fn, 0)


def _stage_b_emulate(cos_flat, thr_flat, x_flat):
    cos = cos_flat.reshape(BM, N)
    t = thr_flat.reshape(BM, 1)
    mask_gt = cos > t
    mask_eq = cos == t
    fill = K - jnp.sum(mask_gt, axis=1, keepdims=True)
    eqrank = jnp.cumsum(mask_eq.astype(jnp.int32), axis=1)
    sel = mask_gt | (mask_eq & (eqrank <= fill))
    idx = jnp.argsort(jnp.where(sel, 0, 1), axis=1, stable=True)[:, :K]
    vals = jnp.take_along_axis(cos, idx, axis=1)
    x = x_flat.reshape(B, N)
    xg = jax.vmap(lambda xb, ib: xb[ib])(x, idx.reshape(B, M * K))
    return vals.reshape(BM * K), xg.reshape(BM * K)


def _stage_b(cos2d, thr_flat, x_flat):
    if _IP:
        return _stage_b_emulate(cos2d.reshape(BM * N), thr_flat, x_flat)
    mesh = plsc.VectorSubcoreMesh(core_axis_name="c", subcore_axis_name="s",
                                  num_cores=_NC, num_subcores=_NS)
    f = pl.kernel(
        _stage_b_body,
        out_type=(jax.ShapeDtypeStruct((BM * K,), jnp.float32),
                  jax.ShapeDtypeStruct((BM * K,), jnp.float32)),
        mesh=mesh,
        scratch_types=[
            pltpu.VMEM((_RB, N), jnp.float32),
            pltpu.VMEM((_RB, N), jnp.float32),
            pltpu.VMEM((N,), jnp.float32),
            pltpu.VMEM((_RPW,), jnp.float32),
            pltpu.VMEM((N,), jnp.int32),
            pltpu.VMEM((N + 16,), jnp.int32),
            pltpu.VMEM((K + 16,), jnp.int32),
            pltpu.VMEM((_RB * K,), jnp.float32),
            pltpu.VMEM((_RB * K,), jnp.float32),
            pltpu.SemaphoreType.DMA,
            pltpu.SemaphoreType.DMA,
        ],
        compiler_params=pltpu.CompilerParams(needs_layout_passes=False),
        interpret=_IP,
    )
    return f(cos2d, thr_flat, x_flat)


MB_C = 512


def _stage_c_body(vals_ref, xg_ref, w1_ref, b1_ref, w2_ref, wp_ref, bp_ref,
                  h_ref):
    vals = vals_ref[...]
    xg = xg_ref[...]
    z = jnp.clip(vals, -1.0, 1.0)
    d = jnp.arctan2(jnp.sqrt(jnp.maximum((1.0 - z) * (1.0 + z), 0.0)), z)
    w1 = w1_ref[...]
    b1 = b1_ref[...]
    w2 = w2_ref[...]
    xgb = _bf(xg)
    db = _bf(d)
    w1b = _bf(w1)
    w2b = _bf(w2)
    logits = jnp.zeros_like(xg)
    for j in range(128):
        hj = jnp.maximum(xgb * w1b[0, j] + db * w1b[1, j] + b1[0, j], 0.0)
        logits = logits + _bf(hj) * w2b[0, j]
    lmax = jnp.max(logits, axis=1, keepdims=True)
    e = jnp.exp(logits - lmax)
    w = e / jnp.sum(e, axis=1, keepdims=True)
    pooled = jnp.sum(w * xg, axis=1, keepdims=True)
    h_ref[...] = jnp.maximum(pooled * wp_ref[...] + bp_ref[...], 0.0)


def _stage_c(vals, xg, att_W1, att_b1, att_W2, att_b2, Wp, bp):
    del att_b2
    b1 = att_b1.reshape(1, 128)
    return pl.pallas_call(
        _stage_c_body,
        grid=(BM // MB_C,),
        in_specs=[
            pl.BlockSpec((MB_C, K), lambda i: (i, 0)),
            pl.BlockSpec((MB_C, K), lambda i: (i, 0)),
            pl.BlockSpec((2, 128), lambda i: (0, 0)),
            pl.BlockSpec((1, 128), lambda i: (0, 0)),
            pl.BlockSpec((1, 128), lambda i: (0, 0)),
            pl.BlockSpec((1, 256), lambda i: (0, 0)),
            pl.BlockSpec((1, 256), lambda i: (0, 0)),
        ],
        out_specs=pl.BlockSpec((MB_C, 256), lambda i: (i, 0)),
        out_shape=jax.ShapeDtypeStruct((BM, 256), jnp.float32),
        interpret=_IP,
    )(vals, xg, att_W1, b1, att_W2.reshape(1, 128), Wp, bp.reshape(1, 256))


MB_D = 256
_BIGI = np.int32(2**30)


def _stage_d_body(pix_ref, pixt_ref, a_ref):
    blk = pl.program_id(0)
    a0 = pix_ref[:, 0:1]
    a1 = pix_ref[:, 1:2]
    a2 = pix_ref[:, 2:3]
    c0 = pixt_ref[0:1, :]
    c1 = pixt_ref[1:2, :]
    c2 = pixt_ref[2:3, :]
    d2 = (a0 - c0) ** 2 + (a1 - c1) ** 2 + (a2 - c2) ** 2
    rows = blk * MB_D + lax.broadcasted_iota(jnp.int32, (MB_D, M), 0)
    cols = lax.broadcasted_iota(jnp.int32, (MB_D, M), 1)
    d2 = jnp.where(rows == cols, jnp.inf, d2)
    nd2 = -d2
    a = jnp.zeros((MB_D, M), jnp.float32)
    for _ in range(GK):
        vmax = jnp.max(nd2, axis=1, keepdims=True)
        cand = jnp.where(nd2 == vmax, cols, _BIGI)
        imin = jnp.min(cand, axis=1, keepdims=True)
        hit = cols == imin
        a = a + jnp.where(hit, 1.0, 0.0)
        nd2 = jnp.where(hit, -jnp.inf, nd2)
    a_ref[...] = a.astype(jnp.bfloat16)


def _stage_d(pix, pixt):
    return pl.pallas_call(
        _stage_d_body,
        grid=(M // MB_D,),
        in_specs=[
            pl.BlockSpec((MB_D, 3), lambda i: (i, 0)),
            pl.BlockSpec((3, M), lambda i: (0, 0)),
        ],
        out_specs=pl.BlockSpec((MB_D, M), lambda i: (i, 0)),
        out_shape=jax.ShapeDtypeStruct((M, M), jnp.bfloat16),
        interpret=_IP,
    )(pix, pixt)


MB_E = 1024


def _gnn_layer_body(hfull_ref, hrows_ref, a_ref, wrel_ref, brel_ref,
                    wroot_ref, out_ref):
    h = hfull_ref[0]
    a = a_ref[...]
    hi = h.astype(jnp.bfloat16)
    lo = (h - hi.astype(jnp.float32)).astype(jnp.bfloat16)
    agg = (jnp.dot(a, hi, preferred_element_type=jnp.float32) +
           jnp.dot(a, lo, preferred_element_type=jnp.float32))
    out_ref[0] = jnp.maximum(
        jnp.dot(agg.astype(jnp.bfloat16), wrel_ref[...].astype(jnp.bfloat16),
                preferred_element_type=jnp.float32)
        + brel_ref[...]
        + jnp.dot(hrows_ref[0].astype(jnp.bfloat16),
                  wroot_ref[...].astype(jnp.bfloat16),
                  preferred_element_type=jnp.float32),
        0.0)


def _gnn_layer(h, a, wrel, brel, wroot):
    return pl.pallas_call(
        _gnn_layer_body,
        grid=(B, M // MB_E),
        in_specs=[
            pl.BlockSpec((1, M, 256), lambda b, j: (b, 0, 0)),
            pl.BlockSpec((1, MB_E, 256), lambda b, j: (b, j, 0)),
            pl.BlockSpec((MB_E, M), lambda b, j: (j, 0)),
            pl.BlockSpec((256, 256), lambda b, j: (0, 0)),
            pl.BlockSpec((1, 256), lambda b, j: (0, 0)),
            pl.BlockSpec((256, 256), lambda b, j: (0, 0)),
        ],
        out_specs=pl.BlockSpec((1, MB_E, 256), lambda b, j: (b, j, 0)),
        out_shape=jax.ShapeDtypeStruct((B, M, 256), jnp.float32),
        interpret=_IP,
    )(h, h, a, wrel, brel, wroot)


def _head_body(h_ref, wo1_ref, bo1_ref, wo2_ref, bo2_ref, out_ref):
    gf = jnp.mean(h_ref[0], axis=0, keepdims=True)
    g1 = jnp.maximum(
        jnp.dot(gf.astype(jnp.bfloat16), wo1_ref[...].astype(jnp.bfloat16),
                preferred_element_type=jnp.float32)
        + bo1_ref[...], 0.0)
    out_ref[0] = (jnp.dot(g1.astype(jnp.bfloat16),
                          wo2_ref[...].astype(jnp.bfloat16),
                          preferred_element_type=jnp.float32)
                  + bo2_ref[...])


def _stage_e(h, a, wrel, brel, wroot, Wo1, bo1, Wo2, bo2):
    for l in range(3):
        h = _gnn_layer(h, a, wrel[l], brel[l].reshape(1, 256), wroot[l])
    return pl.pallas_call(
        _head_body,
        grid=(B,),
        in_specs=[
            pl.BlockSpec((1, M, 256), lambda b: (b, 0, 0)),
            pl.BlockSpec((256, 256), lambda b: (0, 0)),
            pl.BlockSpec((1, 256), lambda b: (0, 0)),
            pl.BlockSpec((256, 200), lambda b: (0, 0)),
            pl.BlockSpec((1, 200), lambda b: (0, 0)),
        ],
        out_specs=pl.BlockSpec((1, 1, 200), lambda b: (b, 0, 0)),
        out_shape=jax.ShapeDtypeStruct((B, 1, 200), jnp.float32),
        interpret=_IP,
    )(h, Wo1, bo1, Wo2, bo2)


def kernel(xxx, pix_theta_phi, att_W1, att_b1, att_W2, att_b2, Wp, bp,
           Wrel0, brel0, Wroot0, Wrel1, brel1, Wroot1, Wrel2, brel2, Wroot2,
           Wo1, bo1, Wo2, bo2):
    th_l = xxx[:, 0, :]
    ph_l = xxx[:, 1, :]
    st_l = jnp.sin(th_l)
    r_losT = jnp.stack([st_l * jnp.cos(ph_l), st_l * jnp.sin(ph_l),
                        jnp.cos(th_l)], axis=1)
    st_p = jnp.sin(pix_theta_phi[:, 0])
    r_pix = jnp.stack([st_p * jnp.cos(pix_theta_phi[:, 1]),
                       st_p * jnp.sin(pix_theta_phi[:, 1]),
                       jnp.cos(pix_theta_phi[:, 0])], axis=-1)

    cos, thr = _stage_a(r_pix, r_losT)
    x_flat = xxx[:, 2, :].reshape(B * N)
    vals_f, xg_f = _stage_b(cos, thr.reshape(BM), x_flat)
    vals = vals_f.reshape(BM, K)
    xg = xg_f.reshape(BM, K)
    h = _stage_c(vals, xg, att_W1, att_b1, att_W2, att_b2, Wp, bp)
    a = _stage_d(r_pix, r_pix.T)
    wrel = jnp.stack([Wrel0, Wrel1, Wrel2])
    brel = jnp.stack([brel0, brel1, brel2])
    wroot = jnp.stack([Wroot0, Wroot1, Wroot2])
    out = _stage_e(h.reshape(B, M, 256), a, wrel, brel, wroot,
                   Wo1, bo1.reshape(1, 256), Wo2, bo2.reshape(1, 200))
    return out.reshape(B, 200)
